# baseline (speedup 1.0000x reference)
import functools

import jax
import jax.numpy as jnp
from jax import lax
from jax.experimental import pallas as pl
from jax.experimental.pallas import tpu as pltpu
from jax.experimental.pallas import tpu_sc as plsc

NUM_USER = 100000
NUM_ITEM = 100000
EMBED = 64
B = 4096
WEIGHT_DECAY = 0.0001

NC = 2
NS = 16
NW = NC * NS
BPW = B // NW
NCHUNK = BPW // 16

HALF = 50048
TW = 2176
TSTEPS = HALF // TW


def _transpose_body(in1_ref, in2_ref, out_ref):
    cat = jnp.concatenate([in1_ref[...], in2_ref[...]], axis=0)
    eye = jnp.eye(2 * EMBED, dtype=jnp.float32)
    out_ref[...] = lax.dot_general(
        cat, eye, (((0,), (0,)), ((), ())),
        preferred_element_type=jnp.float32)


def _pack_table(tab_t):
    return pl.pallas_call(
        _transpose_body,
        grid=(TSTEPS,),
        in_specs=[
            pl.BlockSpec((EMBED, TW), lambda c: (0, c)),
            pl.BlockSpec((EMBED, TW), lambda c: (0, c + TSTEPS)),
        ],
        out_specs=pl.BlockSpec((TW, 2 * EMBED), lambda c: (c, 0)),
        out_shape=jax.ShapeDtypeStruct((HALF, 2 * EMBED), jnp.float32),
    )(tab_t, tab_t)


def _sc_gather2_body(pos_h, neg_h, si_h, bp_h, bn_h,
                     idx_p, idx_n, rows_p, rows_n, sem_p, sem_n):
    wid = lax.axis_index("s") * NC + lax.axis_index("c")
    base = wid * BPW

    pltpu.sync_copy(pos_h.at[pl.ds(base, BPW)], idx_p)
    pltpu.sync_copy(neg_h.at[pl.ds(base, BPW)], idx_n)

    def fold(c, _):
        k0 = c * 16
        for ref in (idx_p, idx_n):
            v = ref[pl.ds(k0, 16)]
            ref[pl.ds(k0, 16)] = jnp.where(v >= HALF, v - HALF, v)
        return _

    lax.fori_loop(0, NCHUNK, fold, None)

    cp = pltpu.async_copy(si_h.at[idx_p], rows_p, sem_p)
    cn = pltpu.async_copy(si_h.at[idx_n], rows_n, sem_n)
    cp.wait()
    pltpu.sync_copy(rows_p, bp_h.at[pl.ds(base, BPW)])
    cn.wait()
    pltpu.sync_copy(rows_n, bn_h.at[pl.ds(base, BPW)])


def _sc_gather1_body(user_h, su_h, bu_h, idx_u, rows_u, sem_u):
    wid = lax.axis_index("s") * NC + lax.axis_index("c")
    base = wid * BPW
    pltpu.sync_copy(user_h.at[pl.ds(base, BPW)], idx_u)

    def fold(c, _):
        k0 = c * 16
        v = idx_u[pl.ds(k0, 16)]
        idx_u[pl.ds(k0, 16)] = jnp.where(v >= HALF, v - HALF, v)
        return _

    lax.fori_loop(0, NCHUNK, fold, None)
    pltpu.async_copy(su_h.at[idx_u], rows_u, sem_u).wait()
    pltpu.sync_copy(rows_u, bu_h.at[pl.ds(base, BPW)])


_SC_MESH = dict(core_axis_name="c", subcore_axis_name="s",
                num_cores=NC, num_subcores=NS)


@functools.cache
def _sc_gather2():
    return pl.kernel(
        _sc_gather2_body,
        out_type=[jax.ShapeDtypeStruct((B, 2 * EMBED), jnp.float32)] * 2,
        mesh=plsc.VectorSubcoreMesh(**_SC_MESH),
        scratch_types=[
            pltpu.VMEM((BPW,), jnp.int32),
            pltpu.VMEM((BPW,), jnp.int32),
            pltpu.VMEM((BPW, 2 * EMBED), jnp.float32),
            pltpu.VMEM((BPW, 2 * EMBED), jnp.float32),
            pltpu.SemaphoreType.DMA,
            pltpu.SemaphoreType.DMA,
        ],
    )


@functools.cache
def _sc_gather1():
    return pl.kernel(
        _sc_gather1_body,
        out_type=jax.ShapeDtypeStruct((B, 2 * EMBED), jnp.float32),
        mesh=plsc.VectorSubcoreMesh(**_SC_MESH),
        scratch_types=[
            pltpu.VMEM((BPW,), jnp.int32),
            pltpu.VMEM((BPW, 2 * EMBED), jnp.float32),
            pltpu.SemaphoreType.DMA,
        ],
    )


def _final_body(user_ref, pos_ref, neg_ref, bu_ref, bp_ref, bn_ref,
                ue_ref, pe_ref, ne_ref, loss_ref, nlp_ref, reg_ref):
    def unpack(idx_1d, buf):
        h = idx_1d[...].reshape(B, 1) >= HALF
        sel = jnp.where(h, buf[:, EMBED:], buf[:, :EMBED])
        return jnp.transpose(sel, (1, 0))

    ue = unpack(user_ref, bu_ref)
    pe = unpack(pos_ref, bp_ref)
    ne = unpack(neg_ref, bn_ref)
    ue_ref[...] = ue
    pe_ref[...] = pe
    ne_ref[...] = ne
    pos_out = jnp.sum(ue * pe, axis=0, keepdims=True)
    neg_out = jnp.sum(ue * ne, axis=0, keepdims=True)
    out = pos_out - neg_out
    log_prob = jnp.sum(jax.nn.log_sigmoid(out))
    reg = WEIGHT_DECAY * (jnp.sum(ue * ue) + jnp.sum(pe * pe)
                          + jnp.sum(ne * ne))
    nlp_ref[0, 0] = -log_prob
    reg_ref[0, 0] = reg
    loss_ref[0, 0] = -log_prob + reg


def _tc_final(user, pos, neg, bu, bp, bn):
    return pl.pallas_call(
        _final_body,
        out_shape=[jax.ShapeDtypeStruct((EMBED, B), jnp.float32)] * 3
        + [jax.ShapeDtypeStruct((1, 1), jnp.float32)] * 3,
        out_specs=[pl.BlockSpec((EMBED, B), lambda: (0, 0))] * 3
        + [pl.BlockSpec(memory_space=pltpu.SMEM)] * 3,
    )(user, pos, neg, bu, bp, bn)


def kernel(user, pos, neg, history, history_mask, user_table, item_table):
    si = _pack_table(item_table.T)
    bp, bn = _sc_gather2()(pos, neg, si)
    su = _pack_table(user_table.T)
    bu = _sc_gather1()(user, su)
    uet, pet, net, loss, nlp, reg = _tc_final(user, pos, neg, bu, bp, bn)
    return (loss[0, 0], nlp[0, 0], reg[0, 0], uet.T, pet.T, net.T)


# speedup vs baseline: 1.0859x; 1.0859x over previous
import functools

import jax
import jax.numpy as jnp
from jax import lax
from jax.experimental import pallas as pl
from jax.experimental.pallas import tpu as pltpu
from jax.experimental.pallas import tpu_sc as plsc

NUM_USER = 100000
NUM_ITEM = 100000
EMBED = 64
B = 4096
WEIGHT_DECAY = 0.0001

NC = 2
NS = 16
NW = NC * NS
BPW = B // NW
NCHUNK = BPW // 16

HALF = 50048
TW = 2176
TSTEPS = HALF // TW


def _transpose_body(in1_ref, in2_ref, out_ref):
    cat = jnp.concatenate([in1_ref[...], in2_ref[...]], axis=0)
    eye = jnp.eye(2 * EMBED, dtype=jnp.float32)
    t = lax.dot_general(
        cat, eye, (((0,), (0,)), ((), ())),
        preferred_element_type=jnp.float32)
    out_ref[...] = pltpu.bitcast(t.astype(jnp.bfloat16), jnp.int32)


def _pack_table(tab_t):
    return pl.pallas_call(
        _transpose_body,
        grid=(TSTEPS,),
        in_specs=[
            pl.BlockSpec((EMBED, TW), lambda c: (0, c)),
            pl.BlockSpec((EMBED, TW), lambda c: (0, c + TSTEPS)),
        ],
        out_specs=pl.BlockSpec((TW // 2, 2 * EMBED), lambda c: (c, 0)),
        out_shape=jax.ShapeDtypeStruct((HALF // 2, 2 * EMBED), jnp.int32),
    )(tab_t, tab_t)


def _sc_gather2_body(pos_h, neg_h, si_h, bp_h, bn_h,
                     idx_p, idx_n, rows_p, rows_n, sem_p, sem_n):
    wid = lax.axis_index("s") * NC + lax.axis_index("c")
    base = wid * BPW

    pltpu.sync_copy(pos_h.at[pl.ds(base, BPW)], idx_p)
    pltpu.sync_copy(neg_h.at[pl.ds(base, BPW)], idx_n)

    def fold(c, _):
        k0 = c * 16
        for ref in (idx_p, idx_n):
            v = ref[pl.ds(k0, 16)]
            ref[pl.ds(k0, 16)] = jnp.where(v >= HALF, v - HALF, v) >> 1
        return _

    lax.fori_loop(0, NCHUNK, fold, None)

    cp = pltpu.async_copy(si_h.at[idx_p], rows_p, sem_p)
    cn = pltpu.async_copy(si_h.at[idx_n], rows_n, sem_n)
    cp.wait()
    pltpu.sync_copy(rows_p, bp_h.at[pl.ds(base, BPW)])
    cn.wait()
    pltpu.sync_copy(rows_n, bn_h.at[pl.ds(base, BPW)])


def _sc_gather1_body(user_h, su_h, bu_h, idx_u, rows_u, sem_u):
    wid = lax.axis_index("s") * NC + lax.axis_index("c")
    base = wid * BPW
    pltpu.sync_copy(user_h.at[pl.ds(base, BPW)], idx_u)

    def fold(c, _):
        k0 = c * 16
        v = idx_u[pl.ds(k0, 16)]
        idx_u[pl.ds(k0, 16)] = jnp.where(v >= HALF, v - HALF, v) >> 1
        return _

    lax.fori_loop(0, NCHUNK, fold, None)
    pltpu.async_copy(su_h.at[idx_u], rows_u, sem_u).wait()
    pltpu.sync_copy(rows_u, bu_h.at[pl.ds(base, BPW)])


_SC_MESH = dict(core_axis_name="c", subcore_axis_name="s",
                num_cores=NC, num_subcores=NS)


@functools.cache
def _sc_gather2():
    return pl.kernel(
        _sc_gather2_body,
        out_type=[jax.ShapeDtypeStruct((B, 2 * EMBED), jnp.int32)] * 2,
        mesh=plsc.VectorSubcoreMesh(**_SC_MESH),
        scratch_types=[
            pltpu.VMEM((BPW,), jnp.int32),
            pltpu.VMEM((BPW,), jnp.int32),
            pltpu.VMEM((BPW, 2 * EMBED), jnp.int32),
            pltpu.VMEM((BPW, 2 * EMBED), jnp.int32),
            pltpu.SemaphoreType.DMA,
            pltpu.SemaphoreType.DMA,
        ],
    )


@functools.cache
def _sc_gather1():
    return pl.kernel(
        _sc_gather1_body,
        out_type=jax.ShapeDtypeStruct((B, 2 * EMBED), jnp.int32),
        mesh=plsc.VectorSubcoreMesh(**_SC_MESH),
        scratch_types=[
            pltpu.VMEM((BPW,), jnp.int32),
            pltpu.VMEM((BPW, 2 * EMBED), jnp.int32),
            pltpu.SemaphoreType.DMA,
        ],
    )


def _final_body(user_ref, pos_ref, neg_ref, bu_ref, bp_ref, bn_ref,
                ue_ref, pe_ref, ne_ref, loss_ref, nlp_ref, reg_ref):
    def unpack(idx_1d, buf):
        idx = idx_1d[...].reshape(B, 1)
        h = idx >= HALF
        b = jnp.where(h, idx - HALF, idx)
        p = (b & 1) == 1
        half = jnp.where(h, buf[:, EMBED:], buf[:, :EMBED])
        v = jnp.where(p, half & jnp.int32(-65536), half << 16)
        sel = lax.bitcast_convert_type(v, jnp.float32)
        return jnp.transpose(sel, (1, 0))

    ue = unpack(user_ref, bu_ref)
    pe = unpack(pos_ref, bp_ref)
    ne = unpack(neg_ref, bn_ref)
    ue_ref[...] = ue
    pe_ref[...] = pe
    ne_ref[...] = ne
    pos_out = jnp.sum(ue * pe, axis=0, keepdims=True)
    neg_out = jnp.sum(ue * ne, axis=0, keepdims=True)
    out = pos_out - neg_out
    log_prob = jnp.sum(jax.nn.log_sigmoid(out))
    reg = WEIGHT_DECAY * (jnp.sum(ue * ue) + jnp.sum(pe * pe)
                          + jnp.sum(ne * ne))
    nlp_ref[0, 0] = -log_prob
    reg_ref[0, 0] = reg
    loss_ref[0, 0] = -log_prob + reg


def _tc_final(user, pos, neg, bu, bp, bn):
    return pl.pallas_call(
        _final_body,
        out_shape=[jax.ShapeDtypeStruct((EMBED, B), jnp.float32)] * 3
        + [jax.ShapeDtypeStruct((1, 1), jnp.float32)] * 3,
        out_specs=[pl.BlockSpec((EMBED, B), lambda: (0, 0))] * 3
        + [pl.BlockSpec(memory_space=pltpu.SMEM)] * 3,
    )(user, pos, neg, bu, bp, bn)


def kernel(user, pos, neg, history, history_mask, user_table, item_table):
    si = _pack_table(item_table.T)
    bp, bn = _sc_gather2()(pos, neg, si)
    su = _pack_table(user_table.T)
    bu = _sc_gather1()(user, su)
    uet, pet, net, loss, nlp, reg = _tc_final(user, pos, neg, bu, bp, bn)
    return (loss[0, 0], nlp[0, 0], reg[0, 0], uet.T, pet.T, net.T)


# speedup vs baseline: 1.1777x; 1.0846x over previous
import functools

import jax
import jax.numpy as jnp
from jax import lax
from jax.experimental import pallas as pl
from jax.experimental.pallas import tpu as pltpu
from jax.experimental.pallas import tpu_sc as plsc

NUM_USER = 100000
NUM_ITEM = 100000
EMBED = 64
B = 4096
WEIGHT_DECAY = 0.0001

NC = 2
NS = 16
NW = NC * NS
BPW = B // NW
NCHUNK = BPW // 16

HALF = 50048
TW = 2944
TSTEPS = HALF // TW


def _transpose_body(in1_ref, in2_ref, out_ref):
    cat = jnp.concatenate([in1_ref[...], in2_ref[...]], axis=0)
    eye = jnp.eye(2 * EMBED, dtype=jnp.float32)
    t = lax.dot_general(
        cat, eye, (((0,), (0,)), ((), ())),
        preferred_element_type=jnp.float32)
    out_ref[...] = pltpu.bitcast(t.astype(jnp.bfloat16), jnp.int32)


def _pack_table(tab_t):
    return pl.pallas_call(
        _transpose_body,
        grid=(TSTEPS,),
        in_specs=[
            pl.BlockSpec((EMBED, TW), lambda c: (0, c)),
            pl.BlockSpec((EMBED, TW), lambda c: (0, c + TSTEPS)),
        ],
        out_specs=pl.BlockSpec((TW // 2, 2 * EMBED), lambda c: (c, 0)),
        out_shape=jax.ShapeDtypeStruct((HALF // 2, 2 * EMBED), jnp.int32),
    )(tab_t, tab_t)


def _sc_gather2_body(pos_h, neg_h, si_h, bp_h, bn_h,
                     idx_p, idx_n, rows_p, rows_n, sem_p, sem_n):
    wid = lax.axis_index("s") * NC + lax.axis_index("c")
    base = wid * BPW

    pltpu.sync_copy(pos_h.at[pl.ds(base, BPW)], idx_p)
    pltpu.sync_copy(neg_h.at[pl.ds(base, BPW)], idx_n)

    def fold(c, _):
        k0 = c * 16
        for ref in (idx_p, idx_n):
            v = ref[pl.ds(k0, 16)]
            ref[pl.ds(k0, 16)] = jnp.where(v >= HALF, v - HALF, v) >> 1
        return _

    lax.fori_loop(0, NCHUNK, fold, None)

    cp = pltpu.async_copy(si_h.at[idx_p], rows_p, sem_p)
    cn = pltpu.async_copy(si_h.at[idx_n], rows_n, sem_n)
    cp.wait()
    pltpu.sync_copy(rows_p, bp_h.at[pl.ds(base, BPW)])
    cn.wait()
    pltpu.sync_copy(rows_n, bn_h.at[pl.ds(base, BPW)])


def _sc_gather1_body(user_h, su_h, bu_h, idx_u, rows_u, sem_u):
    wid = lax.axis_index("s") * NC + lax.axis_index("c")
    base = wid * BPW
    pltpu.sync_copy(user_h.at[pl.ds(base, BPW)], idx_u)

    def fold(c, _):
        k0 = c * 16
        v = idx_u[pl.ds(k0, 16)]
        idx_u[pl.ds(k0, 16)] = jnp.where(v >= HALF, v - HALF, v) >> 1
        return _

    lax.fori_loop(0, NCHUNK, fold, None)
    pltpu.async_copy(su_h.at[idx_u], rows_u, sem_u).wait()
    pltpu.sync_copy(rows_u, bu_h.at[pl.ds(base, BPW)])


_SC_MESH = dict(core_axis_name="c", subcore_axis_name="s",
                num_cores=NC, num_subcores=NS)


@functools.cache
def _sc_gather2():
    return pl.kernel(
        _sc_gather2_body,
        out_type=[jax.ShapeDtypeStruct((B, 2 * EMBED), jnp.int32)] * 2,
        mesh=plsc.VectorSubcoreMesh(**_SC_MESH),
        scratch_types=[
            pltpu.VMEM((BPW,), jnp.int32),
            pltpu.VMEM((BPW,), jnp.int32),
            pltpu.VMEM((BPW, 2 * EMBED), jnp.int32),
            pltpu.VMEM((BPW, 2 * EMBED), jnp.int32),
            pltpu.SemaphoreType.DMA,
            pltpu.SemaphoreType.DMA,
        ],
    )


@functools.cache
def _sc_gather1():
    return pl.kernel(
        _sc_gather1_body,
        out_type=jax.ShapeDtypeStruct((B, 2 * EMBED), jnp.int32),
        mesh=plsc.VectorSubcoreMesh(**_SC_MESH),
        scratch_types=[
            pltpu.VMEM((BPW,), jnp.int32),
            pltpu.VMEM((BPW, 2 * EMBED), jnp.int32),
            pltpu.SemaphoreType.DMA,
        ],
    )


def _final_body(user_ref, pos_ref, neg_ref, bu_ref, bp_ref, bn_ref,
                ue_ref, pe_ref, ne_ref, loss_ref, nlp_ref, reg_ref):
    def unpack(idx_1d, buf):
        idx = idx_1d[...].reshape(B, 1)
        h = idx >= HALF
        b = jnp.where(h, idx - HALF, idx)
        p = (b & 1) == 1
        half = jnp.where(h, buf[:, EMBED:], buf[:, :EMBED])
        v = jnp.where(p, half & jnp.int32(-65536), half << 16)
        sel = lax.bitcast_convert_type(v, jnp.float32)
        return jnp.transpose(sel, (1, 0))

    ue = unpack(user_ref, bu_ref)
    pe = unpack(pos_ref, bp_ref)
    ne = unpack(neg_ref, bn_ref)
    ue_ref[...] = ue
    pe_ref[...] = pe
    ne_ref[...] = ne
    pos_out = jnp.sum(ue * pe, axis=0, keepdims=True)
    neg_out = jnp.sum(ue * ne, axis=0, keepdims=True)
    out = pos_out - neg_out
    log_prob = jnp.sum(jax.nn.log_sigmoid(out))
    reg = WEIGHT_DECAY * (jnp.sum(ue * ue) + jnp.sum(pe * pe)
                          + jnp.sum(ne * ne))
    nlp_ref[0, 0] = -log_prob
    reg_ref[0, 0] = reg
    loss_ref[0, 0] = -log_prob + reg


def _tc_final(user, pos, neg, bu, bp, bn):
    return pl.pallas_call(
        _final_body,
        out_shape=[jax.ShapeDtypeStruct((EMBED, B), jnp.float32)] * 3
        + [jax.ShapeDtypeStruct((1, 1), jnp.float32)] * 3,
        out_specs=[pl.BlockSpec((EMBED, B), lambda: (0, 0))] * 3
        + [pl.BlockSpec(memory_space=pltpu.SMEM)] * 3,
    )(user, pos, neg, bu, bp, bn)


def kernel(user, pos, neg, history, history_mask, user_table, item_table):
    si = _pack_table(item_table.T)
    bp, bn = _sc_gather2()(pos, neg, si)
    su = _pack_table(user_table.T)
    bu = _sc_gather1()(user, su)
    uet, pet, net, loss, nlp, reg = _tc_final(user, pos, neg, bu, bp, bn)
    return (loss[0, 0], nlp[0, 0], reg[0, 0], uet.T, pet.T, net.T)


# speedup vs baseline: 1.3598x; 1.1546x over previous
import functools

import jax
import jax.numpy as jnp
from jax import lax
from jax.experimental import pallas as pl
from jax.experimental.pallas import tpu as pltpu
from jax.experimental.pallas import tpu_sc as plsc

NUM_USER = 100000
NUM_ITEM = 100000
EMBED = 64
B = 4096
WEIGHT_DECAY = 0.0001

NC = 2
NS = 16
NW = NC * NS
BPW = B // NW
NCHUNK = BPW // 16

HALF = 50176
TW = 6272
TSTEPS = HALF // TW


def _transpose_body(in1_ref, in2_ref, out_ref):
    cat = jnp.concatenate([in1_ref[...], in2_ref[...]], axis=0)
    eye = jnp.eye(2 * EMBED, dtype=jnp.float32)
    t = lax.dot_general(
        cat, eye, (((0,), (0,)), ((), ())),
        preferred_element_type=jnp.float32)
    out_ref[...] = pltpu.bitcast(t.astype(jnp.bfloat16), jnp.int32)


def _pack_table(tab_t):
    return pl.pallas_call(
        _transpose_body,
        grid=(TSTEPS,),
        in_specs=[
            pl.BlockSpec((EMBED, TW), lambda c: (0, c)),
            pl.BlockSpec((EMBED, TW), lambda c: (0, c + TSTEPS)),
        ],
        out_specs=pl.BlockSpec((TW // 2, 2 * EMBED), lambda c: (c, 0)),
        out_shape=jax.ShapeDtypeStruct((HALF // 2, 2 * EMBED), jnp.int32),
    )(tab_t, tab_t)


def _sc_gather2_body(pos_h, neg_h, si_h, bp_h, bn_h,
                     idx_p, idx_n, rows_p, rows_n, sem_p, sem_n):
    wid = lax.axis_index("s") * NC + lax.axis_index("c")
    base = wid * BPW

    pltpu.sync_copy(pos_h.at[pl.ds(base, BPW)], idx_p)
    pltpu.sync_copy(neg_h.at[pl.ds(base, BPW)], idx_n)

    def fold(c, _):
        k0 = c * 16
        for ref in (idx_p, idx_n):
            v = ref[pl.ds(k0, 16)]
            ref[pl.ds(k0, 16)] = jnp.where(v >= HALF, v - HALF, v) >> 1
        return _

    lax.fori_loop(0, NCHUNK, fold, None)

    cp = pltpu.async_copy(si_h.at[idx_p], rows_p, sem_p)
    cn = pltpu.async_copy(si_h.at[idx_n], rows_n, sem_n)
    cp.wait()
    pltpu.sync_copy(rows_p, bp_h.at[pl.ds(base, BPW)])
    cn.wait()
    pltpu.sync_copy(rows_n, bn_h.at[pl.ds(base, BPW)])


def _sc_gather1_body(user_h, su_h, bu_h, idx_u, rows_u, sem_u):
    wid = lax.axis_index("s") * NC + lax.axis_index("c")
    base = wid * BPW
    pltpu.sync_copy(user_h.at[pl.ds(base, BPW)], idx_u)

    def fold(c, _):
        k0 = c * 16
        v = idx_u[pl.ds(k0, 16)]
        idx_u[pl.ds(k0, 16)] = jnp.where(v >= HALF, v - HALF, v) >> 1
        return _

    lax.fori_loop(0, NCHUNK, fold, None)
    pltpu.async_copy(su_h.at[idx_u], rows_u, sem_u).wait()
    pltpu.sync_copy(rows_u, bu_h.at[pl.ds(base, BPW)])


_SC_MESH = dict(core_axis_name="c", subcore_axis_name="s",
                num_cores=NC, num_subcores=NS)


@functools.cache
def _sc_gather2():
    return pl.kernel(
        _sc_gather2_body,
        out_type=[jax.ShapeDtypeStruct((B, 2 * EMBED), jnp.int32)] * 2,
        mesh=plsc.VectorSubcoreMesh(**_SC_MESH),
        scratch_types=[
            pltpu.VMEM((BPW,), jnp.int32),
            pltpu.VMEM((BPW,), jnp.int32),
            pltpu.VMEM((BPW, 2 * EMBED), jnp.int32),
            pltpu.VMEM((BPW, 2 * EMBED), jnp.int32),
            pltpu.SemaphoreType.DMA,
            pltpu.SemaphoreType.DMA,
        ],
    )


@functools.cache
def _sc_gather1():
    return pl.kernel(
        _sc_gather1_body,
        out_type=jax.ShapeDtypeStruct((B, 2 * EMBED), jnp.int32),
        mesh=plsc.VectorSubcoreMesh(**_SC_MESH),
        scratch_types=[
            pltpu.VMEM((BPW,), jnp.int32),
            pltpu.VMEM((BPW, 2 * EMBED), jnp.int32),
            pltpu.SemaphoreType.DMA,
        ],
    )


def _final_body(user_ref, pos_ref, neg_ref, bu_ref, bp_ref, bn_ref,
                ue_ref, pe_ref, ne_ref, loss_ref, nlp_ref, reg_ref):
    def unpack(idx_1d, buf):
        idx = idx_1d[...].reshape(B, 1)
        h = idx >= HALF
        b = jnp.where(h, idx - HALF, idx)
        p = (b & 1) == 1
        half = jnp.where(h, buf[:, EMBED:], buf[:, :EMBED])
        v = jnp.where(p, half & jnp.int32(-65536), half << 16)
        sel = lax.bitcast_convert_type(v, jnp.float32)
        return jnp.transpose(sel, (1, 0))

    ue = unpack(user_ref, bu_ref)
    pe = unpack(pos_ref, bp_ref)
    ne = unpack(neg_ref, bn_ref)
    ue_ref[...] = ue
    pe_ref[...] = pe
    ne_ref[...] = ne
    pos_out = jnp.sum(ue * pe, axis=0, keepdims=True)
    neg_out = jnp.sum(ue * ne, axis=0, keepdims=True)
    out = pos_out - neg_out
    log_prob = jnp.sum(jax.nn.log_sigmoid(out))
    reg = WEIGHT_DECAY * (jnp.sum(ue * ue) + jnp.sum(pe * pe)
                          + jnp.sum(ne * ne))
    nlp_ref[0, 0] = -log_prob
    reg_ref[0, 0] = reg
    loss_ref[0, 0] = -log_prob + reg


def _tc_final(user, pos, neg, bu, bp, bn):
    return pl.pallas_call(
        _final_body,
        out_shape=[jax.ShapeDtypeStruct((EMBED, B), jnp.float32)] * 3
        + [jax.ShapeDtypeStruct((1, 1), jnp.float32)] * 3,
        out_specs=[pl.BlockSpec((EMBED, B), lambda: (0, 0))] * 3
        + [pl.BlockSpec(memory_space=pltpu.SMEM)] * 3,
    )(user, pos, neg, bu, bp, bn)


def kernel(user, pos, neg, history, history_mask, user_table, item_table):
    si = _pack_table(item_table.T)
    bp, bn = _sc_gather2()(pos, neg, si)
    su = _pack_table(user_table.T)
    bu = _sc_gather1()(user, su)
    uet, pet, net, loss, nlp, reg = _tc_final(user, pos, neg, bu, bp, bn)
    return (loss[0, 0], nlp[0, 0], reg[0, 0], uet.T, pet.T, net.T)


# speedup vs baseline: 1.4366x; 1.0565x over previous
import functools

import jax
import jax.numpy as jnp
from jax import lax
from jax.experimental import pallas as pl
from jax.experimental.pallas import tpu as pltpu
from jax.experimental.pallas import tpu_sc as plsc

NUM_USER = 100000
NUM_ITEM = 100000
EMBED = 64
B = 4096
WEIGHT_DECAY = 0.0001

NC = 2
NS = 16
NW = NC * NS
BPW = B // NW
NCHUNK = BPW // 16

HALF = 50176
TW = 12544
TSTEPS = HALF // TW


def _transpose_body(in1_ref, in2_ref, out_ref):
    cat = jnp.concatenate([in1_ref[...], in2_ref[...]], axis=0)
    eye = jnp.eye(2 * EMBED, dtype=jnp.float32)
    t = lax.dot_general(
        cat, eye, (((0,), (0,)), ((), ())),
        preferred_element_type=jnp.float32)
    out_ref[...] = pltpu.bitcast(t.astype(jnp.bfloat16), jnp.int32)


def _pack_table(tab_t):
    return pl.pallas_call(
        _transpose_body,
        grid=(TSTEPS,),
        in_specs=[
            pl.BlockSpec((EMBED, TW), lambda c: (0, c)),
            pl.BlockSpec((EMBED, TW), lambda c: (0, c + TSTEPS)),
        ],
        out_specs=pl.BlockSpec((TW // 2, 2 * EMBED), lambda c: (c, 0)),
        out_shape=jax.ShapeDtypeStruct((HALF // 2, 2 * EMBED), jnp.int32),
    )(tab_t, tab_t)


def _sc_gather2_body(pos_h, neg_h, si_h, bp_h, bn_h,
                     idx_p, idx_n, rows_p, rows_n, sem_p, sem_n):
    wid = lax.axis_index("s") * NC + lax.axis_index("c")
    base = wid * BPW

    pltpu.sync_copy(pos_h.at[pl.ds(base, BPW)], idx_p)
    pltpu.sync_copy(neg_h.at[pl.ds(base, BPW)], idx_n)

    def fold(c, _):
        k0 = c * 16
        for ref in (idx_p, idx_n):
            v = ref[pl.ds(k0, 16)]
            ref[pl.ds(k0, 16)] = jnp.where(v >= HALF, v - HALF, v) >> 1
        return _

    lax.fori_loop(0, NCHUNK, fold, None)

    cp = pltpu.async_copy(si_h.at[idx_p], rows_p, sem_p)
    cn = pltpu.async_copy(si_h.at[idx_n], rows_n, sem_n)
    cp.wait()
    pltpu.sync_copy(rows_p, bp_h.at[pl.ds(base, BPW)])
    cn.wait()
    pltpu.sync_copy(rows_n, bn_h.at[pl.ds(base, BPW)])


def _sc_gather1_body(user_h, su_h, bu_h, idx_u, rows_u, sem_u):
    wid = lax.axis_index("s") * NC + lax.axis_index("c")
    base = wid * BPW
    pltpu.sync_copy(user_h.at[pl.ds(base, BPW)], idx_u)

    def fold(c, _):
        k0 = c * 16
        v = idx_u[pl.ds(k0, 16)]
        idx_u[pl.ds(k0, 16)] = jnp.where(v >= HALF, v - HALF, v) >> 1
        return _

    lax.fori_loop(0, NCHUNK, fold, None)
    pltpu.async_copy(su_h.at[idx_u], rows_u, sem_u).wait()
    pltpu.sync_copy(rows_u, bu_h.at[pl.ds(base, BPW)])


_SC_MESH = dict(core_axis_name="c", subcore_axis_name="s",
                num_cores=NC, num_subcores=NS)


@functools.cache
def _sc_gather2():
    return pl.kernel(
        _sc_gather2_body,
        out_type=[jax.ShapeDtypeStruct((B, 2 * EMBED), jnp.int32)] * 2,
        mesh=plsc.VectorSubcoreMesh(**_SC_MESH),
        scratch_types=[
            pltpu.VMEM((BPW,), jnp.int32),
            pltpu.VMEM((BPW,), jnp.int32),
            pltpu.VMEM((BPW, 2 * EMBED), jnp.int32),
            pltpu.VMEM((BPW, 2 * EMBED), jnp.int32),
            pltpu.SemaphoreType.DMA,
            pltpu.SemaphoreType.DMA,
        ],
    )


@functools.cache
def _sc_gather1():
    return pl.kernel(
        _sc_gather1_body,
        out_type=jax.ShapeDtypeStruct((B, 2 * EMBED), jnp.int32),
        mesh=plsc.VectorSubcoreMesh(**_SC_MESH),
        scratch_types=[
            pltpu.VMEM((BPW,), jnp.int32),
            pltpu.VMEM((BPW, 2 * EMBED), jnp.int32),
            pltpu.SemaphoreType.DMA,
        ],
    )


def _final_body(user_ref, pos_ref, neg_ref, bu_ref, bp_ref, bn_ref,
                ue_ref, pe_ref, ne_ref, loss_ref, nlp_ref, reg_ref):
    def unpack(idx_1d, buf):
        idx = idx_1d[...].reshape(B, 1)
        h = idx >= HALF
        b = jnp.where(h, idx - HALF, idx)
        p = (b & 1) == 1
        half = jnp.where(h, buf[:, EMBED:], buf[:, :EMBED])
        v = jnp.where(p, half & jnp.int32(-65536), half << 16)
        sel = lax.bitcast_convert_type(v, jnp.float32)
        return jnp.transpose(sel, (1, 0))

    ue = unpack(user_ref, bu_ref)
    pe = unpack(pos_ref, bp_ref)
    ne = unpack(neg_ref, bn_ref)
    ue_ref[...] = ue
    pe_ref[...] = pe
    ne_ref[...] = ne
    pos_out = jnp.sum(ue * pe, axis=0, keepdims=True)
    neg_out = jnp.sum(ue * ne, axis=0, keepdims=True)
    out = pos_out - neg_out
    log_prob = jnp.sum(jax.nn.log_sigmoid(out))
    reg = WEIGHT_DECAY * (jnp.sum(ue * ue) + jnp.sum(pe * pe)
                          + jnp.sum(ne * ne))
    nlp_ref[0, 0] = -log_prob
    reg_ref[0, 0] = reg
    loss_ref[0, 0] = -log_prob + reg


def _tc_final(user, pos, neg, bu, bp, bn):
    return pl.pallas_call(
        _final_body,
        out_shape=[jax.ShapeDtypeStruct((EMBED, B), jnp.float32)] * 3
        + [jax.ShapeDtypeStruct((1, 1), jnp.float32)] * 3,
        out_specs=[pl.BlockSpec((EMBED, B), lambda: (0, 0))] * 3
        + [pl.BlockSpec(memory_space=pltpu.SMEM)] * 3,
    )(user, pos, neg, bu, bp, bn)


def kernel(user, pos, neg, history, history_mask, user_table, item_table):
    si = _pack_table(item_table.T)
    bp, bn = _sc_gather2()(pos, neg, si)
    su = _pack_table(user_table.T)
    bu = _sc_gather1()(user, su)
    uet, pet, net, loss, nlp, reg = _tc_final(user, pos, neg, bu, bp, bn)
    return (loss[0, 0], nlp[0, 0], reg[0, 0], uet.T, pet.T, net.T)
